# initial kernel scaffold (unmeasured)
import jax
import jax.numpy as jnp
from jax import lax
from jax.experimental import pallas as pl
from jax.experimental.pallas import tpu as pltpu

N_DEV = 4


def kernel(x, w_mat):
    m_per, k_dim = x.shape
    _, n_dim = w_mat.shape
    n_per = n_dim // N_DEV
    m_tot = m_per * N_DEV

    def body(
        x_ref,
        w_ref,
        out_ref,
        y_ref,
        q_ref,
        qin_ref,
        amax_src,
        amax_rcv,
        send_sems,
        recv_sems,
        am_send_sems,
        am_recv_sems,
    ):
        my = lax.axis_index("i")

        barrier = pltpu.get_barrier_semaphore()
        for k in range(1, N_DEV):
            pl.semaphore_signal(
                barrier, inc=1,
                device_id=((my + k) % N_DEV,),
                device_id_type=pl.DeviceIdType.MESH,
            )
        pl.semaphore_wait(barrier, N_DEV - 1)

        y_ref[...] = jnp.dot(
            x_ref[...], w_ref[...], preferred_element_type=jnp.float32
        )

        amax = jnp.max(jnp.abs(y_ref[...]))
        amax_src[...] = jnp.full((8, 128), amax, jnp.float32)
        am_rdmas = []
        for k in range(1, N_DEV):
            tgt = (my + k) % N_DEV
            r = pltpu.make_async_remote_copy(
                src_ref=amax_src,
                dst_ref=amax_rcv.at[k - 1],
                send_sem=am_send_sems.at[k - 1],
                recv_sem=am_recv_sems.at[k - 1],
                device_id=(tgt,),
                device_id_type=pl.DeviceIdType.MESH,
            )
            r.start()
            am_rdmas.append(r)
        g_amax = amax
        for k in range(1, N_DEV):
            am_rdmas[k - 1].wait()
            g_amax = jnp.maximum(g_amax, jnp.max(amax_rcv[k - 1]))
        scale = g_amax / 448.0

        for j in range(N_DEV):
            q_ref[j] = jnp.clip(
                y_ref[:, j * n_per:(j + 1) * n_per] / scale, -448.0, 448.0
            ).astype(jnp.float8_e4m3fn)

        rdmas = []
        for k in range(1, N_DEV):
            tgt = (my + k) % N_DEV
            r = pltpu.make_async_remote_copy(
                src_ref=q_ref.at[tgt],
                dst_ref=qin_ref.at[k - 1],
                send_sem=send_sems.at[k - 1],
                recv_sem=recv_sems.at[k - 1],
                device_id=(tgt,),
                device_id_type=pl.DeviceIdType.MESH,
            )
            r.start()
            rdmas.append(r)

        for j in range(N_DEV):
            @pl.when(j == my)
            def _():
                out_ref[j * m_per:(j + 1) * m_per, :] = (
                    q_ref[j].astype(jnp.float32) * scale
                )

        for k in range(1, N_DEV):
            rdmas[k - 1].wait()
            src_dev = (my - k) % N_DEV
            out_ref[pl.ds(src_dev * m_per, m_per), :] = (
                qin_ref[k - 1].astype(jnp.float32) * scale
            )

    return pl.pallas_call(
        body,
        out_shape=jax.ShapeDtypeStruct((m_tot, n_per), jnp.float32),
        in_specs=[
            pl.BlockSpec(memory_space=pltpu.VMEM),
            pl.BlockSpec(memory_space=pltpu.VMEM),
        ],
        out_specs=pl.BlockSpec(memory_space=pltpu.VMEM),
        scratch_shapes=[
            pltpu.VMEM((m_per, n_dim), jnp.float32),
            pltpu.VMEM((N_DEV, m_per, n_per), jnp.float8_e4m3fn),
            pltpu.VMEM((N_DEV - 1, m_per, n_per), jnp.float8_e4m3fn),
            pltpu.VMEM((8, 128), jnp.float32),
            pltpu.VMEM((N_DEV - 1, 8, 128), jnp.float32),
            pltpu.SemaphoreType.DMA((N_DEV - 1,)),
            pltpu.SemaphoreType.DMA((N_DEV - 1,)),
            pltpu.SemaphoreType.DMA((N_DEV - 1,)),
            pltpu.SemaphoreType.DMA((N_DEV - 1,)),
        ],
        compiler_params=pltpu.CompilerParams(
            collective_id=0,
            vmem_limit_bytes=100 * 1024 * 1024,
        ),
    )(x, w_mat)


# baseline (device time: 74798 ns/iter reference)
import jax
import jax.numpy as jnp
from jax import lax
from jax.experimental import pallas as pl
from jax.experimental.pallas import tpu as pltpu

N_DEV = 4
NB = 256


def kernel(x, w_mat):
    m_per, k_dim = x.shape
    _, n_dim = w_mat.shape
    n_per = n_dim // N_DEV
    m_tot = m_per * N_DEV
    n_blocks = n_dim // NB

    def body(
        x_ref,
        w_hbm,
        out_ref,
        y_ref,
        w_buf,
        q_ref,
        qin_ref,
        amax_src,
        amax_rcv,
        w_sems,
        send_sems,
        recv_sems,
        am_send_sems,
        am_recv_sems,
    ):
        my = lax.axis_index("i")

        barrier = pltpu.get_barrier_semaphore()
        for k in range(1, N_DEV):
            pl.semaphore_signal(
                barrier, inc=1,
                device_id=((my + k) % N_DEV,),
                device_id_type=pl.DeviceIdType.MESH,
            )
        pl.semaphore_wait(barrier, N_DEV - 1)

        def w_copy(b):
            return pltpu.make_async_copy(
                w_hbm.at[:, b * NB:(b + 1) * NB],
                w_buf.at[b % 2],
                w_sems.at[b % 2],
            )

        w_copy(0).start()
        w_copy(1).start()
        for b in range(n_blocks):
            w_copy(b).wait()
            y_ref[:, b * NB:(b + 1) * NB] = jnp.dot(
                x_ref[...], w_buf[b % 2], preferred_element_type=jnp.float32
            )
            if b + 2 < n_blocks:
                w_copy(b + 2).start()

        amax = jnp.max(jnp.abs(y_ref[...]))
        amax_src[...] = jnp.full((8, 128), amax, jnp.float32)
        am_rdmas = []
        for k in range(1, N_DEV):
            tgt = (my + k) % N_DEV
            r = pltpu.make_async_remote_copy(
                src_ref=amax_src,
                dst_ref=amax_rcv.at[k - 1],
                send_sem=am_send_sems.at[k - 1],
                recv_sem=am_recv_sems.at[k - 1],
                device_id=(tgt,),
                device_id_type=pl.DeviceIdType.MESH,
            )
            r.start()
            am_rdmas.append(r)
        g_amax = amax
        for k in range(1, N_DEV):
            am_rdmas[k - 1].wait()
            g_amax = jnp.maximum(g_amax, jnp.max(amax_rcv[k - 1]))
        scale = g_amax / 448.0

        for j in range(N_DEV):
            q_ref[j] = jnp.clip(
                y_ref[:, j * n_per:(j + 1) * n_per] / scale, -448.0, 448.0
            ).astype(jnp.float8_e4m3fn)

        rdmas = []
        for k in range(1, N_DEV):
            tgt = (my + k) % N_DEV
            r = pltpu.make_async_remote_copy(
                src_ref=q_ref.at[tgt],
                dst_ref=qin_ref.at[k - 1],
                send_sem=send_sems.at[k - 1],
                recv_sem=recv_sems.at[k - 1],
                device_id=(tgt,),
                device_id_type=pl.DeviceIdType.MESH,
            )
            r.start()
            rdmas.append(r)

        for j in range(N_DEV):
            @pl.when(j == my)
            def _():
                out_ref[j * m_per:(j + 1) * m_per, :] = (
                    q_ref[j].astype(jnp.float32) * scale
                )

        for k in range(1, N_DEV):
            rdmas[k - 1].wait()
            src_dev = (my - k) % N_DEV
            out_ref[pl.ds(src_dev * m_per, m_per), :] = (
                qin_ref[k - 1].astype(jnp.float32) * scale
            )

    return pl.pallas_call(
        body,
        out_shape=jax.ShapeDtypeStruct((m_tot, n_per), jnp.float32),
        in_specs=[
            pl.BlockSpec(memory_space=pltpu.VMEM),
            pl.BlockSpec(memory_space=pl.ANY),
        ],
        out_specs=pl.BlockSpec(memory_space=pltpu.VMEM),
        scratch_shapes=[
            pltpu.VMEM((m_per, n_dim), jnp.float32),
            pltpu.VMEM((2, k_dim, NB), jnp.float32),
            pltpu.VMEM((N_DEV, m_per, n_per), jnp.float8_e4m3fn),
            pltpu.VMEM((N_DEV - 1, m_per, n_per), jnp.float8_e4m3fn),
            pltpu.VMEM((8, 128), jnp.float32),
            pltpu.VMEM((N_DEV - 1, 8, 128), jnp.float32),
            pltpu.SemaphoreType.DMA((2,)),
            pltpu.SemaphoreType.DMA((N_DEV - 1,)),
            pltpu.SemaphoreType.DMA((N_DEV - 1,)),
            pltpu.SemaphoreType.DMA((N_DEV - 1,)),
            pltpu.SemaphoreType.DMA((N_DEV - 1,)),
        ],
        compiler_params=pltpu.CompilerParams(
            collective_id=0,
            vmem_limit_bytes=100 * 1024 * 1024,
        ),
    )(x, w_mat)


# device time: 74151 ns/iter; 1.0087x vs baseline; 1.0087x over previous
import jax
import jax.numpy as jnp
from jax import lax
from jax.experimental import pallas as pl
from jax.experimental.pallas import tpu as pltpu

N_DEV = 4
NB = 256


def kernel(x, w_mat):
    m_per, k_dim = x.shape
    _, n_dim = w_mat.shape
    n_per = n_dim // N_DEV
    m_tot = m_per * N_DEV
    n_blocks = n_dim // NB

    def body(
        x_ref,
        w_hbm,
        out_ref,
        xbf_ref,
        w_buf,
        wbf_buf,
        y_ref,
        q_ref,
        qin_ref,
        amax_src,
        amax_rcv,
        w_sems,
        send_sems,
        recv_sems,
        am_send_sems,
        am_recv_sems,
    ):
        my = lax.axis_index("i")

        def w_copy(b):
            return pltpu.make_async_copy(
                w_hbm.at[:, b * NB:(b + 1) * NB],
                w_buf.at[b % 2],
                w_sems.at[b % 2],
            )

        w_copy(0).start()
        w_copy(1).start()

        barrier = pltpu.get_barrier_semaphore()
        for k in range(1, N_DEV):
            pl.semaphore_signal(
                barrier, inc=1,
                device_id=((my + k) % N_DEV,),
                device_id_type=pl.DeviceIdType.MESH,
            )

        xbf_ref[...] = x_ref[...].astype(jnp.bfloat16)

        amax = jnp.float32(0.0)
        for b in range(n_blocks):
            w_copy(b).wait()
            wbf_buf[b % 2] = w_buf[b % 2].astype(jnp.bfloat16)
            if b + 2 < n_blocks:
                w_copy(b + 2).start()
            yv = jnp.dot(
                xbf_ref[...], wbf_buf[b % 2],
                preferred_element_type=jnp.float32,
            )
            y_ref[:, b * NB:(b + 1) * NB] = yv
            amax = jnp.maximum(amax, jnp.max(jnp.abs(yv)))

        pl.semaphore_wait(barrier, N_DEV - 1)

        amax_src[...] = jnp.full((8, 128), amax, jnp.float32)
        am_rdmas = []
        for k in range(1, N_DEV):
            tgt = (my + k) % N_DEV
            r = pltpu.make_async_remote_copy(
                src_ref=amax_src,
                dst_ref=amax_rcv.at[k - 1],
                send_sem=am_send_sems.at[k - 1],
                recv_sem=am_recv_sems.at[k - 1],
                device_id=(tgt,),
                device_id_type=pl.DeviceIdType.MESH,
            )
            r.start()
            am_rdmas.append(r)
        g_amax = amax
        for k in range(1, N_DEV):
            am_rdmas[k - 1].wait()
            g_amax = jnp.maximum(g_amax, jnp.max(amax_rcv[k - 1]))
        scale = g_amax / 448.0

        for j in range(N_DEV):
            q_ref[j] = jnp.clip(
                y_ref[:, j * n_per:(j + 1) * n_per] / scale, -448.0, 448.0
            ).astype(jnp.float8_e4m3fn)

        rdmas = []
        for k in range(1, N_DEV):
            tgt = (my + k) % N_DEV
            r = pltpu.make_async_remote_copy(
                src_ref=q_ref.at[tgt],
                dst_ref=qin_ref.at[k - 1],
                send_sem=send_sems.at[k - 1],
                recv_sem=recv_sems.at[k - 1],
                device_id=(tgt,),
                device_id_type=pl.DeviceIdType.MESH,
            )
            r.start()
            rdmas.append(r)

        for j in range(N_DEV):
            @pl.when(j == my)
            def _():
                out_ref[j * m_per:(j + 1) * m_per, :] = (
                    q_ref[j].astype(jnp.float32) * scale
                )

        for k in range(1, N_DEV):
            rdmas[k - 1].wait()
            src_dev = (my - k) % N_DEV
            out_ref[pl.ds(src_dev * m_per, m_per), :] = (
                qin_ref[k - 1].astype(jnp.float32) * scale
            )

    return pl.pallas_call(
        body,
        out_shape=jax.ShapeDtypeStruct((m_tot, n_per), jnp.float32),
        in_specs=[
            pl.BlockSpec(memory_space=pltpu.VMEM),
            pl.BlockSpec(memory_space=pl.ANY),
        ],
        out_specs=pl.BlockSpec(memory_space=pltpu.VMEM),
        scratch_shapes=[
            pltpu.VMEM((m_per, k_dim), jnp.bfloat16),
            pltpu.VMEM((2, k_dim, NB), jnp.float32),
            pltpu.VMEM((2, k_dim, NB), jnp.bfloat16),
            pltpu.VMEM((m_per, n_dim), jnp.float32),
            pltpu.VMEM((N_DEV, m_per, n_per), jnp.float8_e4m3fn),
            pltpu.VMEM((N_DEV - 1, m_per, n_per), jnp.float8_e4m3fn),
            pltpu.VMEM((8, 128), jnp.float32),
            pltpu.VMEM((N_DEV - 1, 8, 128), jnp.float32),
            pltpu.SemaphoreType.DMA((2,)),
            pltpu.SemaphoreType.DMA((N_DEV - 1,)),
            pltpu.SemaphoreType.DMA((N_DEV - 1,)),
            pltpu.SemaphoreType.DMA((N_DEV - 1,)),
            pltpu.SemaphoreType.DMA((N_DEV - 1,)),
        ],
        compiler_params=pltpu.CompilerParams(
            collective_id=0,
            vmem_limit_bytes=100 * 1024 * 1024,
        ),
    )(x, w_mat)
